# baseline (speedup 1.0000x reference)
import functools

import jax
import jax.numpy as jnp
from jax import lax
from jax.experimental import pallas as pl
from jax.experimental.pallas import tpu as pltpu
from jax.experimental.pallas import tpu_sc as plsc

_R = 256
_K = 262144
_G = 32
_M = 64
_S = 100
_NT = 16
_NC = 2
_SPC = _M // _NC


def _prep_kernel(w_ref, p_ref, xft_ref, inn_ref, init_ref, xf0_ref,
                 t_ref, xb_ref, keep_ref, b0_ref):
    iota_p = lax.broadcasted_iota(jnp.int32, (1, 256), 1)
    iota_j = lax.broadcasted_iota(jnp.int32, (8, 1), 0)
    bits_t = ((jnp.broadcast_to(iota_p, (8, 256)) >> iota_j) & 1).astype(jnp.float32)
    w = w_ref[...]
    p = p_ref[...]
    for g in range(_G):
        wpg = (w[:, 8 * g:8 * g + 8] * p[:, 8 * g:8 * g + 8]).astype(jnp.float32)
        tg = lax.dot_general(wpg, bits_t, (((1,), (0,)), ((), ())),
                             precision=lax.Precision.HIGHEST,
                             preferred_element_type=jnp.float32)
        tgi = tg.astype(jnp.int32).reshape(16, 16, 256)
        packed = tgi[:, 0:8, :] | (tgi[:, 8:16, :] << 16)
        t_ref[:, g, :, :] = packed

    inn = inn_ref[...]
    iota_g = lax.broadcasted_iota(jnp.int32, (_G, 1), 0)
    bval = jnp.where((inn // 8) == iota_g, 1 << (inn % 8), 0)
    keep_ref[...] = 255 - jnp.sum(bval, axis=1, keepdims=True)

    xft = xft_ref[...]
    acc = jnp.zeros((_M, _G, _S), jnp.int32)
    for pos in range(16):
        acc = acc + xft[:, pos:pos + 1, :] * bval[:, pos:pos + 1][None]
    xb_ref[...] = acc

    iota8 = lax.broadcasted_iota(jnp.int32, (1, 8), 1)
    initb = jnp.sum(init_ref[...] << iota8, axis=1, keepdims=True)
    xf0 = xf0_ref[...]
    xb0 = jnp.zeros((_G, _M), jnp.int32)
    for pos in range(16):
        xb0 = xb0 + bval[:, pos:pos + 1] * xf0[pos:pos + 1, :]
    b0_ref[...] = (initb & keep_ref[...]) | xb0


def _sc_body(t_hbm, xb_hbm, b0_hbm, keep_hbm, lut_hbm, rw_hbm, rb_hbm, out_hbm,
             t_v, xb_v, ball, pub, b0v, keep_t, idxbuf, vals, rw_v, rb_v,
             part_v, pub2, outbuf, sh_bytes, sh_part, sem):
    c = lax.axis_index("c")
    t = lax.axis_index("s")

    pltpu.sync_copy(t_hbm.at[t], t_v)
    pltpu.sync_copy(xb_hbm.at[t, c], xb_v)
    pltpu.sync_copy(keep_hbm.at[t], keep_t)
    pltpu.sync_copy(rw_hbm.at[pl.ds(t * 16, 16)], rw_v)
    pltpu.sync_copy(rb_hbm, rb_v)
    pltpu.sync_copy(b0_hbm.at[t], b0v)
    pltpu.sync_copy(b0v.at[pl.ds(c * 64, 64)], sh_bytes.at[pl.ds(t * 64, 64)])
    plsc.subcore_barrier()

    def step(j, carry):
        buf = j & 1
        pltpu.sync_copy(sh_bytes.at[pl.ds(buf * 1024, 1024)], ball)

        cps = []
        for b in range(2):
            def g_body(g, accs):
                bytev = ball[pl.ds(g * 32 + b * 16, 16)]
                base = g * 2048
                new = []
                for jl in range(8):
                    idx = bytev + (base + jl * 256)
                    v = plsc.load_gather(t_v, [idx])
                    new.append(accs[jl] + (v & 0xFFFF))
                    new.append(accs[8 + jl] + lax.shift_right_logical(v, 16))
                return tuple(new[0::2]) + tuple(new[1::2])

            zero = jnp.zeros((16,), jnp.int32)
            accs = lax.fori_loop(0, _G, g_body, (zero,) * 16, unroll=8)
            for jl in range(8):
                for half, s in ((0, accs[jl]), (1, accs[8 + jl])):
                    f = b * 256 + (jl + 8 * half) * 16
                    n = t * 16 + jl + 8 * half
                    base = (n // 8) * 2097152 + (n % 8) * 128
                    idxbuf[f // 128, pl.ds(f % 128, 16)] = (
                        base + (lax.shift_right_logical(s, 7) << 10) + (s & 127))
            cps += [pltpu.async_copy(lut_hbm.at[idxbuf.at[q]], vals.at[q], sem)
                    for q in (2 * b, 2 * b + 1)]

        jn = jnp.minimum(j + 1, _S - 1)
        kv = keep_t[pl.ds(0, 16)]
        for b in range(2):
            cps[2 * b].wait()
            cps[2 * b + 1].wait()
            for gl in range(2):
                kp = kv[gl]
                lutbyte = jnp.zeros((16,), jnp.int32)
                for jl in range(8):
                    f = b * 256 + (jl + 8 * gl) * 16
                    lutbyte = lutbyte | (vals[f // 128, pl.ds(f % 128, 16)] << jl)
                xv = xb_v[pl.ds(gl * 3200 + jn * 32 + b * 16, 16)]
                pub[pl.ds(gl * 32 + b * 16, 16)] = (lutbyte & kp) | xv
        nbuf = 1 - buf
        pltpu.sync_copy(pub, sh_bytes.at[pl.ds(nbuf * 1024 + t * 64, 64)])
        plsc.subcore_barrier()
        return carry

    lax.fori_loop(0, _S, step, 0)

    for cl in range(2):
        for b in range(2):
            acc = jnp.zeros((16,), jnp.float32)
            for jj in range(16):
                f = b * 256 + jj * 16
                v = vals[f // 128, pl.ds(f % 128, 16)].astype(jnp.float32)
                acc = acc + v * rw_v[jj, pl.ds(0, 16)][cl]
            pub2[pl.ds(cl * 32 + b * 16, 16)] = acc
    pltpu.sync_copy(pub2, sh_part.at[pl.ds(t * 64, 64)])
    plsc.subcore_barrier()

    @pl.when(t == 0)
    def _():
        pltpu.sync_copy(sh_part, part_v)
        iot = lax.iota(jnp.int32, 16)
        for cl in range(2):
            for b in range(2):
                acc = jnp.zeros((16,), jnp.float32)
                for tt in range(_NT):
                    acc = acc + part_v[pl.ds(tt * 64 + cl * 32 + b * 16, 16)]
                acc = acc + rb_v[pl.ds(0, 16)][cl]
                plsc.store_scatter(outbuf, [iot * 2 + (b * 32 + cl)], acc)
        pltpu.sync_copy(outbuf, out_hbm.at[c])


def _make_sc_rollout():
    return pl.kernel(
        _sc_body,
        out_type=jax.ShapeDtypeStruct((_NC, 128), jnp.float32),
        mesh=plsc.VectorSubcoreMesh(core_axis_name="c", subcore_axis_name="s",
                                    num_cores=_NC, num_subcores=_NT),
        compiler_params=pltpu.CompilerParams(needs_layout_passes=False),
        scratch_types=[
        pltpu.VMEM((8 * _G * 256,), jnp.int32),
        pltpu.VMEM((2 * _S * _SPC,), jnp.int32),
        pltpu.VMEM((_G * _SPC,), jnp.int32),
        pltpu.VMEM((2 * _SPC,), jnp.int32),
        pltpu.VMEM((128,), jnp.int32),
        pltpu.VMEM((128,), jnp.int32),
        pltpu.VMEM((4, 128), jnp.int32),
        pltpu.VMEM((4, 128), jnp.int32),
        pltpu.VMEM((16, 128), jnp.float32),
        pltpu.VMEM((128,), jnp.float32),
        pltpu.VMEM((_NT * 64,), jnp.float32),
        pltpu.VMEM((64,), jnp.float32),
        pltpu.VMEM((128,), jnp.float32),
            pltpu.VMEM_SHARED((2 * _G * _SPC,), jnp.int32),
            pltpu.VMEM_SHARED((_NT * 64,), jnp.float32),
            pltpu.SemaphoreType.DMA,
        ],
    )


def kernel(x, lut_tensor, initial_reservoir, W_reservoir, primes, input_nodes,
           readout_w, readout_b):
    w_i = W_reservoir.astype(jnp.int32)
    primes2 = primes.reshape(1, _R).astype(jnp.int32)
    xi = x.astype(jnp.int32).reshape(_M, _S, 16)
    xft = xi.transpose(0, 2, 1)
    xf0 = xi[:, 0, :].T
    inn = input_nodes.reshape(1, 16).astype(jnp.int32)
    init2 = initial_reservoir.astype(jnp.int32).reshape(_G, 8)

    t4, xbyte, keep, b0 = pl.pallas_call(
        _prep_kernel,
        out_shape=[
            jax.ShapeDtypeStruct((16, _G, 8, 256), jnp.int32),
            jax.ShapeDtypeStruct((_M, _G, _S), jnp.int32),
            jax.ShapeDtypeStruct((_G, 1), jnp.int32),
            jax.ShapeDtypeStruct((_G, _M), jnp.int32),
        ],
    )(w_i, primes2, xft, inn, init2, xf0)

    t2 = t4.reshape(16, 65536)
    xb3 = (xbyte.reshape(_NC, _SPC, _NT, 2, _S)
           .transpose(2, 0, 3, 4, 1).reshape(_NT, _NC, 2 * _S * _SPC))
    b03 = (b0.reshape(_NT, 2, _NC, _SPC)
           .transpose(0, 2, 1, 3).reshape(_NT, 128))
    keep_pad = jnp.pad(keep.reshape(_NT, 2), ((0, 0), (0, 126)))
    rw_pad = jnp.pad(readout_w.T.astype(jnp.float32), ((0, 0), (0, 126)))
    rb_pad = jnp.pad(readout_b.astype(jnp.float32), (0, 126))
    lutf = (lut_tensor.reshape(32, 8, 2048, 128).transpose(0, 2, 1, 3)
            .reshape(_R * _K))

    outf = _make_sc_rollout()(t2, xb3, b03, keep_pad, lutf, rw_pad, rb_pad)
    return outf[:, :64].reshape(_M, 2)


# speedup vs baseline: 1.3323x; 1.3323x over previous
import functools

import jax
import jax.numpy as jnp
from jax import lax
from jax.experimental import pallas as pl
from jax.experimental.pallas import tpu as pltpu
from jax.experimental.pallas import tpu_sc as plsc

_R = 256
_K = 262144
_G = 32
_M = 64
_S = 100
_NT = 16
_NC = 2
_SPC = _M // _NC


def _prep_kernel(w_ref, p_ref, xft_ref, inn_ref, init_ref, xf0_ref,
                 t_ref, xb_ref, keep_ref, b0_ref):
    iota_p = lax.broadcasted_iota(jnp.int32, (1, 256), 1)
    iota_j = lax.broadcasted_iota(jnp.int32, (8, 1), 0)
    bits_t = ((jnp.broadcast_to(iota_p, (8, 256)) >> iota_j) & 1).astype(jnp.float32)
    w = w_ref[...]
    p = p_ref[...]
    for g in range(_G):
        wpg = (w[:, 8 * g:8 * g + 8] * p[:, 8 * g:8 * g + 8]).astype(jnp.float32)
        tg = lax.dot_general(wpg, bits_t, (((1,), (0,)), ((), ())),
                             precision=lax.Precision.HIGHEST,
                             preferred_element_type=jnp.float32)
        tgi = tg.astype(jnp.int32).reshape(16, 16, 256)
        packed = tgi[:, 0:8, :] | (tgi[:, 8:16, :] << 16)
        t_ref[:, g, :, :] = packed

    inn = inn_ref[...]
    iota_g = lax.broadcasted_iota(jnp.int32, (_G, 1), 0)
    bval = jnp.where((inn // 8) == iota_g, 1 << (inn % 8), 0)
    keep_ref[...] = 255 - jnp.sum(bval, axis=1, keepdims=True)

    xft = xft_ref[...]
    acc = jnp.zeros((_M, _G, _S), jnp.int32)
    for pos in range(16):
        acc = acc + xft[:, pos:pos + 1, :] * bval[:, pos:pos + 1][None]
    xb_ref[...] = acc

    iota8 = lax.broadcasted_iota(jnp.int32, (1, 8), 1)
    initb = jnp.sum(init_ref[...] << iota8, axis=1, keepdims=True)
    xf0 = xf0_ref[...]
    xb0 = jnp.zeros((_G, _M), jnp.int32)
    for pos in range(16):
        xb0 = xb0 + bval[:, pos:pos + 1] * xf0[pos:pos + 1, :]
    b0_ref[...] = (initb & keep_ref[...]) | xb0


def _sc_body(t_hbm, xb_hbm, b0_hbm, keep_hbm, lut_hbm, rw_hbm, rb_hbm, out_hbm,
             t_v, xb_v, ball, pub, b0v, keep_t, idxbuf, vals, rw_v, rb_v,
             part_v, pub2, outbuf, sh_bytes, sh_part, sem):
    c = lax.axis_index("c")
    t = lax.axis_index("s")

    pltpu.sync_copy(t_hbm.at[t], t_v)
    pltpu.sync_copy(xb_hbm.at[t, c], xb_v)
    pltpu.sync_copy(keep_hbm.at[t], keep_t)
    pltpu.sync_copy(rw_hbm.at[pl.ds(t * 16, 16)], rw_v)
    pltpu.sync_copy(rb_hbm, rb_v)
    pltpu.sync_copy(b0_hbm.at[t], b0v)
    pltpu.sync_copy(b0v.at[pl.ds(c * 64, 64)], sh_bytes.at[pl.ds(t * 64, 64)])
    plsc.subcore_barrier()

    def step(j, carry):
        buf = j & 1
        pltpu.sync_copy(sh_bytes.at[pl.ds(buf * 1024, 1024)], ball)

        cps = []
        for b in range(2):
            def g_body(g, accs):
                bytev = ball[pl.ds(g * 32 + b * 16, 16)]
                base = g * 2048
                new = []
                for jl in range(8):
                    idx = bytev + (base + jl * 256)
                    v = plsc.load_gather(t_v, [idx])
                    new.append(accs[jl] + (v & 0xFFFF))
                    new.append(accs[8 + jl] + lax.shift_right_logical(v, 16))
                return tuple(new[0::2]) + tuple(new[1::2])

            zero = jnp.zeros((16,), jnp.int32)
            accs = lax.fori_loop(0, _G, g_body, (zero,) * 16, unroll=4)
            for jl in range(8):
                for half, s in ((0, accs[jl]), (1, accs[8 + jl])):
                    f = b * 256 + (jl + 8 * half) * 16
                    n = t * 16 + jl + 8 * half
                    base = (n // 8) * 2097152 + (n % 8) * 128
                    idxbuf[f // 128, pl.ds(f % 128, 16)] = (
                        base + (lax.shift_right_logical(s, 7) << 10) + (s & 127))
            cps += [pltpu.async_copy(lut_hbm.at[idxbuf.at[q]], vals.at[q], sem)
                    for q in (2 * b, 2 * b + 1)]

        jn = jnp.minimum(j + 1, _S - 1)
        kv = keep_t[pl.ds(0, 16)]
        for b in range(2):
            cps[2 * b].wait()
            cps[2 * b + 1].wait()
            for gl in range(2):
                kp = kv[gl]
                lutbyte = jnp.zeros((16,), jnp.int32)
                for jl in range(8):
                    f = b * 256 + (jl + 8 * gl) * 16
                    lutbyte = lutbyte | (vals[f // 128, pl.ds(f % 128, 16)] << jl)
                xv = xb_v[pl.ds(gl * 3200 + jn * 32 + b * 16, 16)]
                pub[pl.ds(gl * 32 + b * 16, 16)] = (lutbyte & kp) | xv
        nbuf = 1 - buf
        pltpu.sync_copy(pub, sh_bytes.at[pl.ds(nbuf * 1024 + t * 64, 64)])
        plsc.subcore_barrier()
        return carry

    lax.fori_loop(0, _S, step, 0)

    for cl in range(2):
        for b in range(2):
            acc = jnp.zeros((16,), jnp.float32)
            for jj in range(16):
                f = b * 256 + jj * 16
                v = vals[f // 128, pl.ds(f % 128, 16)].astype(jnp.float32)
                acc = acc + v * rw_v[jj, pl.ds(0, 16)][cl]
            pub2[pl.ds(cl * 32 + b * 16, 16)] = acc
    pltpu.sync_copy(pub2, sh_part.at[pl.ds(t * 64, 64)])
    plsc.subcore_barrier()

    @pl.when(t == 0)
    def _():
        pltpu.sync_copy(sh_part, part_v)
        iot = lax.iota(jnp.int32, 16)
        for cl in range(2):
            for b in range(2):
                acc = jnp.zeros((16,), jnp.float32)
                for tt in range(_NT):
                    acc = acc + part_v[pl.ds(tt * 64 + cl * 32 + b * 16, 16)]
                acc = acc + rb_v[pl.ds(0, 16)][cl]
                plsc.store_scatter(outbuf, [iot * 2 + (b * 32 + cl)], acc)
        pltpu.sync_copy(outbuf, out_hbm.at[c])


def _make_sc_rollout():
    return pl.kernel(
        _sc_body,
        out_type=jax.ShapeDtypeStruct((_NC, 128), jnp.float32),
        mesh=plsc.VectorSubcoreMesh(core_axis_name="c", subcore_axis_name="s",
                                    num_cores=_NC, num_subcores=_NT),
        compiler_params=pltpu.CompilerParams(needs_layout_passes=False),
        scratch_types=[
        pltpu.VMEM((8 * _G * 256,), jnp.int32),
        pltpu.VMEM((2 * _S * _SPC,), jnp.int32),
        pltpu.VMEM((_G * _SPC,), jnp.int32),
        pltpu.VMEM((2 * _SPC,), jnp.int32),
        pltpu.VMEM((128,), jnp.int32),
        pltpu.VMEM((128,), jnp.int32),
        pltpu.VMEM((4, 128), jnp.int32),
        pltpu.VMEM((4, 128), jnp.int32),
        pltpu.VMEM((16, 128), jnp.float32),
        pltpu.VMEM((128,), jnp.float32),
        pltpu.VMEM((_NT * 64,), jnp.float32),
        pltpu.VMEM((64,), jnp.float32),
        pltpu.VMEM((128,), jnp.float32),
            pltpu.VMEM_SHARED((2 * _G * _SPC,), jnp.int32),
            pltpu.VMEM_SHARED((_NT * 64,), jnp.float32),
            pltpu.SemaphoreType.DMA,
        ],
    )


def kernel(x, lut_tensor, initial_reservoir, W_reservoir, primes, input_nodes,
           readout_w, readout_b):
    w_i = W_reservoir.astype(jnp.int32)
    primes2 = primes.reshape(1, _R).astype(jnp.int32)
    xi = x.astype(jnp.int32).reshape(_M, _S, 16)
    xft = xi.transpose(0, 2, 1)
    xf0 = xi[:, 0, :].T
    inn = input_nodes.reshape(1, 16).astype(jnp.int32)
    init2 = initial_reservoir.astype(jnp.int32).reshape(_G, 8)

    t4, xbyte, keep, b0 = pl.pallas_call(
        _prep_kernel,
        out_shape=[
            jax.ShapeDtypeStruct((16, _G, 8, 256), jnp.int32),
            jax.ShapeDtypeStruct((_M, _G, _S), jnp.int32),
            jax.ShapeDtypeStruct((_G, 1), jnp.int32),
            jax.ShapeDtypeStruct((_G, _M), jnp.int32),
        ],
    )(w_i, primes2, xft, inn, init2, xf0)

    t2 = t4.reshape(16, 65536)
    xb3 = (xbyte.reshape(_NC, _SPC, _NT, 2, _S)
           .transpose(2, 0, 3, 4, 1).reshape(_NT, _NC, 2 * _S * _SPC))
    b03 = (b0.reshape(_NT, 2, _NC, _SPC)
           .transpose(0, 2, 1, 3).reshape(_NT, 128))
    keep_pad = jnp.pad(keep.reshape(_NT, 2), ((0, 0), (0, 126)))
    rw_pad = jnp.pad(readout_w.T.astype(jnp.float32), ((0, 0), (0, 126)))
    rb_pad = jnp.pad(readout_b.astype(jnp.float32), (0, 126))
    lutf = (lut_tensor.reshape(32, 8, 2048, 128).transpose(0, 2, 1, 3)
            .reshape(_R * _K))

    outf = _make_sc_rollout()(t2, xb3, b03, keep_pad, lutf, rw_pad, rb_pad)
    return outf[:, :64].reshape(_M, 2)


# speedup vs baseline: 1.3849x; 1.0395x over previous
import functools

import jax
import jax.numpy as jnp
from jax import lax
from jax.experimental import pallas as pl
from jax.experimental.pallas import tpu as pltpu
from jax.experimental.pallas import tpu_sc as plsc

_R = 256
_K = 262144
_G = 32
_M = 64
_S = 100
_NT = 16
_NC = 2
_SPC = _M // _NC


def _prep_kernel(w_ref, p_ref, xft_ref, inn_ref, init_ref, xf0_ref,
                 t_ref, xb_ref, keep_ref, b0_ref):
    iota_p = lax.broadcasted_iota(jnp.int32, (1, 256), 1)
    iota_j = lax.broadcasted_iota(jnp.int32, (8, 1), 0)
    bits_t = ((jnp.broadcast_to(iota_p, (8, 256)) >> iota_j) & 1).astype(jnp.float32)
    w = w_ref[...]
    p = p_ref[...]
    for g in range(_G):
        wpg = (w[:, 8 * g:8 * g + 8] * p[:, 8 * g:8 * g + 8]).astype(jnp.float32)
        tg = lax.dot_general(wpg, bits_t, (((1,), (0,)), ((), ())),
                             precision=lax.Precision.HIGHEST,
                             preferred_element_type=jnp.float32)
        tgi = tg.astype(jnp.int32).reshape(16, 16, 256)
        packed = tgi[:, 0:8, :] | (tgi[:, 8:16, :] << 16)
        t_ref[:, g, :, :] = packed

    inn = inn_ref[...]
    iota_g = lax.broadcasted_iota(jnp.int32, (_G, 1), 0)
    bval = jnp.where((inn // 8) == iota_g, 1 << (inn % 8), 0)
    keep_ref[...] = 255 - jnp.sum(bval, axis=1, keepdims=True)

    xft = xft_ref[...]
    acc = jnp.zeros((_M, _G, _S), jnp.int32)
    for pos in range(16):
        acc = acc + xft[:, pos:pos + 1, :] * bval[:, pos:pos + 1][None]
    xb_ref[...] = acc

    iota8 = lax.broadcasted_iota(jnp.int32, (1, 8), 1)
    initb = jnp.sum(init_ref[...] << iota8, axis=1, keepdims=True)
    xf0 = xf0_ref[...]
    xb0 = jnp.zeros((_G, _M), jnp.int32)
    for pos in range(16):
        xb0 = xb0 + bval[:, pos:pos + 1] * xf0[pos:pos + 1, :]
    b0_ref[...] = (initb & keep_ref[...]) | xb0


def _sc_body(t_hbm, xb_hbm, b0_hbm, keep_hbm, lut_hbm, rw_hbm, rb_hbm, out_hbm,
             t_v, xb_v, ball, pub, b0v, keep_t, idxbuf, vals, rw_v, rb_v,
             part_v, pub2, outbuf, sh_bytes, sh_part, sem):
    c = lax.axis_index("c")
    t = lax.axis_index("s")

    pltpu.sync_copy(t_hbm.at[t], t_v)
    pltpu.sync_copy(xb_hbm.at[t, c], xb_v)
    pltpu.sync_copy(keep_hbm.at[t], keep_t)
    pltpu.sync_copy(rw_hbm.at[pl.ds(t * 16, 16)], rw_v)
    pltpu.sync_copy(rb_hbm, rb_v)
    pltpu.sync_copy(b0_hbm.at[t], b0v)
    pltpu.sync_copy(b0v.at[pl.ds(c * 64, 64)], sh_bytes.at[pl.ds(t * 64, 64)])
    plsc.subcore_barrier()

    def step(j, carry):
        buf = j & 1
        pltpu.sync_copy(sh_bytes.at[pl.ds(buf * 1024, 1024)], ball)

        cps = []
        for b in range(2):
            def g_body(i, accs):
                g0 = i * 4
                idxs = []
                for k in range(4):
                    bytev = ball[pl.ds((g0 + k) * 32 + b * 16, 16)]
                    idxs.append(bytev + (g0 + k) * 2048)
                new_lo, new_hi = [], []
                for jl in range(8):
                    tref = t_v.at[pl.ds(jl * 256, 63744)]
                    vs = [plsc.load_gather(tref, [idxs[k]]) for k in range(4)]
                    s4 = (vs[0] + vs[1]) + (vs[2] + vs[3])
                    new_lo.append(accs[jl] + (s4 & 0xFFFF))
                    new_hi.append(accs[8 + jl] + lax.shift_right_logical(s4, 16))
                return tuple(new_lo) + tuple(new_hi)

            zero = jnp.zeros((16,), jnp.int32)
            accs = lax.fori_loop(0, 8, g_body, (zero,) * 16)
            for jl in range(8):
                for half, s in ((0, accs[jl]), (1, accs[8 + jl])):
                    f = b * 256 + (jl + 8 * half) * 16
                    n = t * 16 + jl + 8 * half
                    base = (n // 8) * 2097152 + (n % 8) * 128
                    idxbuf[f // 128, pl.ds(f % 128, 16)] = (
                        base + (lax.shift_right_logical(s, 7) << 10) + (s & 127))
            cps += [pltpu.async_copy(lut_hbm.at[idxbuf.at[q]], vals.at[q], sem)
                    for q in (2 * b, 2 * b + 1)]

        jn = jnp.minimum(j + 1, _S - 1)
        kv = keep_t[pl.ds(0, 16)]
        for b in range(2):
            cps[2 * b].wait()
            cps[2 * b + 1].wait()
            for gl in range(2):
                kp = kv[gl]
                lutbyte = jnp.zeros((16,), jnp.int32)
                for jl in range(8):
                    f = b * 256 + (jl + 8 * gl) * 16
                    lutbyte = lutbyte | (vals[f // 128, pl.ds(f % 128, 16)] << jl)
                xv = xb_v[pl.ds(gl * 3200 + jn * 32 + b * 16, 16)]
                pub[pl.ds(gl * 32 + b * 16, 16)] = (lutbyte & kp) | xv
        nbuf = 1 - buf
        pltpu.sync_copy(pub, sh_bytes.at[pl.ds(nbuf * 1024 + t * 64, 64)])
        plsc.subcore_barrier()
        return carry

    lax.fori_loop(0, _S, step, 0)

    for cl in range(2):
        for b in range(2):
            acc = jnp.zeros((16,), jnp.float32)
            for jj in range(16):
                f = b * 256 + jj * 16
                v = vals[f // 128, pl.ds(f % 128, 16)].astype(jnp.float32)
                acc = acc + v * rw_v[jj, pl.ds(0, 16)][cl]
            pub2[pl.ds(cl * 32 + b * 16, 16)] = acc
    pltpu.sync_copy(pub2, sh_part.at[pl.ds(t * 64, 64)])
    plsc.subcore_barrier()

    @pl.when(t == 0)
    def _():
        pltpu.sync_copy(sh_part, part_v)
        iot = lax.iota(jnp.int32, 16)
        for cl in range(2):
            for b in range(2):
                acc = jnp.zeros((16,), jnp.float32)
                for tt in range(_NT):
                    acc = acc + part_v[pl.ds(tt * 64 + cl * 32 + b * 16, 16)]
                acc = acc + rb_v[pl.ds(0, 16)][cl]
                plsc.store_scatter(outbuf, [iot * 2 + (b * 32 + cl)], acc)
        pltpu.sync_copy(outbuf, out_hbm.at[c])


def _make_sc_rollout():
    return pl.kernel(
        _sc_body,
        out_type=jax.ShapeDtypeStruct((_NC, 128), jnp.float32),
        mesh=plsc.VectorSubcoreMesh(core_axis_name="c", subcore_axis_name="s",
                                    num_cores=_NC, num_subcores=_NT),
        compiler_params=pltpu.CompilerParams(needs_layout_passes=False),
        scratch_types=[
        pltpu.VMEM((8 * _G * 256,), jnp.int32),
        pltpu.VMEM((2 * _S * _SPC,), jnp.int32),
        pltpu.VMEM((_G * _SPC,), jnp.int32),
        pltpu.VMEM((2 * _SPC,), jnp.int32),
        pltpu.VMEM((128,), jnp.int32),
        pltpu.VMEM((128,), jnp.int32),
        pltpu.VMEM((4, 128), jnp.int32),
        pltpu.VMEM((4, 128), jnp.int32),
        pltpu.VMEM((16, 128), jnp.float32),
        pltpu.VMEM((128,), jnp.float32),
        pltpu.VMEM((_NT * 64,), jnp.float32),
        pltpu.VMEM((64,), jnp.float32),
        pltpu.VMEM((128,), jnp.float32),
            pltpu.VMEM_SHARED((2 * _G * _SPC,), jnp.int32),
            pltpu.VMEM_SHARED((_NT * 64,), jnp.float32),
            pltpu.SemaphoreType.DMA,
        ],
    )


def kernel(x, lut_tensor, initial_reservoir, W_reservoir, primes, input_nodes,
           readout_w, readout_b):
    w_i = W_reservoir.astype(jnp.int32)
    primes2 = primes.reshape(1, _R).astype(jnp.int32)
    xi = x.astype(jnp.int32).reshape(_M, _S, 16)
    xft = xi.transpose(0, 2, 1)
    xf0 = xi[:, 0, :].T
    inn = input_nodes.reshape(1, 16).astype(jnp.int32)
    init2 = initial_reservoir.astype(jnp.int32).reshape(_G, 8)

    t4, xbyte, keep, b0 = pl.pallas_call(
        _prep_kernel,
        out_shape=[
            jax.ShapeDtypeStruct((16, _G, 8, 256), jnp.int32),
            jax.ShapeDtypeStruct((_M, _G, _S), jnp.int32),
            jax.ShapeDtypeStruct((_G, 1), jnp.int32),
            jax.ShapeDtypeStruct((_G, _M), jnp.int32),
        ],
    )(w_i, primes2, xft, inn, init2, xf0)

    t2 = t4.reshape(16, 65536)
    xb3 = (xbyte.reshape(_NC, _SPC, _NT, 2, _S)
           .transpose(2, 0, 3, 4, 1).reshape(_NT, _NC, 2 * _S * _SPC))
    b03 = (b0.reshape(_NT, 2, _NC, _SPC)
           .transpose(0, 2, 1, 3).reshape(_NT, 128))
    keep_pad = jnp.pad(keep.reshape(_NT, 2), ((0, 0), (0, 126)))
    rw_pad = jnp.pad(readout_w.T.astype(jnp.float32), ((0, 0), (0, 126)))
    rb_pad = jnp.pad(readout_b.astype(jnp.float32), (0, 126))
    lutf = (lut_tensor.reshape(32, 8, 2048, 128).transpose(0, 2, 1, 3)
            .reshape(_R * _K))

    outf = _make_sc_rollout()(t2, xb3, b03, keep_pad, lutf, rw_pad, rb_pad)
    return outf[:, :64].reshape(_M, 2)
